# baseline (speedup 1.0000x reference)
import jax
import jax.numpy as jnp
from jax.experimental import pallas as pl
import jax.experimental.pallas.tpu as pltpu

SEG_LEN = 32
OUT_LEN = 1056
ZERO_ROWS = 1008
ZBLK = 16
NZ = ZERO_ROWS // ZBLK


def _body(seg_hbm, out_hbm, zbuf, dbuf, zsems, dsem):
    in_cp = pltpu.make_async_copy(seg_hbm, dbuf.at[pl.ds(16, SEG_LEN)], dsem)
    in_cp.start()

    zbuf[...] = jnp.zeros_like(zbuf)
    for i in range(NZ):
        pltpu.make_async_copy(
            zbuf, out_hbm.at[pl.ds(i * ZBLK, ZBLK)], zsems.at[i]
        ).start()

    in_cp.wait()
    seg = dbuf[pl.ds(16, SEG_LEN)]
    dbuf[0:16] = seg.reshape(16, 2, *seg.shape[1:])[:, 0]
    data_cp = pltpu.make_async_copy(
        dbuf, out_hbm.at[pl.ds(ZERO_ROWS, 48)], dsem
    )
    data_cp.start()

    for i in range(NZ):
        pltpu.make_async_copy(
            zbuf, out_hbm.at[pl.ds(i * ZBLK, ZBLK)], zsems.at[i]
        ).wait()
    data_cp.wait()


def kernel(current_segment):
    seg_len, batch, emb = current_segment.shape

    return pl.pallas_call(
        _body,
        in_specs=[pl.BlockSpec(memory_space=pltpu.MemorySpace.HBM)],
        out_specs=pl.BlockSpec(memory_space=pltpu.MemorySpace.HBM),
        out_shape=jax.ShapeDtypeStruct(
            (OUT_LEN, batch, emb), current_segment.dtype
        ),
        scratch_shapes=[
            pltpu.VMEM((ZBLK, batch, emb), current_segment.dtype),
            pltpu.VMEM((48, batch, emb), current_segment.dtype),
            pltpu.SemaphoreType.DMA((NZ,)),
            pltpu.SemaphoreType.DMA,
        ],
    )(current_segment)
